# baseline (speedup 1.0000x reference)
import functools

import jax
import jax.numpy as jnp
from jax import lax
from jax.experimental import pallas as pl
from jax.experimental.pallas import tpu as pltpu
from jax.experimental.pallas import tpu_sc as plsc

N = 100000
W = 128
MAXD = 4
L = 16
NC = 2
NS = 16
NW = NC * NS
CHUNK = 160
NBUF = 4
NCHUNKS = N // CHUNK
ITERS = -(-NCHUNKS // NW)
assert ITERS % NBUF == 0

_mesh = plsc.VectorSubcoreMesh(core_axis_name="c", subcore_axis_name="s")


@functools.partial(
    pl.kernel,
    out_type=jax.ShapeDtypeStruct((N, W), jnp.float32),
    mesh=_mesh,
    scratch_types=(
        [pltpu.VMEM((MAXD, W), jnp.float32)]
        + [pltpu.VMEM((CHUNK, W), jnp.float32)] * NBUF
        + [pltpu.VMEM((ITERS * CHUNK,), jnp.int32)]
        + [pltpu.SemaphoreType.DMA] * NBUF
        + [pltpu.SemaphoreType.DMA] * NBUF
        + [pltpu.SemaphoreType.DMA]
    ),
)
def _scale_degree(x_hbm, d_hbm, scale_hbm, out_hbm, wtab, *bufs):
    xbufs = bufs[0:NBUF]
    dball = bufs[NBUF]
    sin = bufs[NBUF + 1:2 * NBUF + 1]
    sout = bufs[2 * NBUF + 1:3 * NBUF + 1]
    sd = bufs[3 * NBUF + 1]
    wid = lax.axis_index("s") * NC + lax.axis_index("c")


    def valid(it):
        return (it * NW + wid) < NCHUNKS

    def in_descr(it, b):
        base = (it * NW + wid) * CHUNK
        return pltpu.make_async_copy(x_hbm.at[pl.ds(base, CHUNK)], xbufs[b], sin[b])

    def d_descr(it_static):
        base = (it_static * NW + wid) * CHUNK
        return pltpu.make_async_copy(d_hbm.at[pl.ds(base, CHUNK)],
                                     dball.at[pl.ds(it_static * CHUNK, CHUNK)], sd)

    def out_descr(it, b):
        base = (it * NW + wid) * CHUNK
        return pltpu.make_async_copy(xbufs[b], out_hbm.at[pl.ds(base, CHUNK)], sout[b])

    def start_in(it, b):
        @pl.when(valid(it))
        def _():
            in_descr(it, b).start()

    start_in(0, 0)
    start_in(1, 1)

    for itc in range(ITERS):
        @pl.when(valid(itc))
        def _(itc=itc):
            d_descr(itc).start()
    for itc in range(ITERS):
        @pl.when(valid(itc))
        def _(itc=itc):
            d_descr(itc).wait()

    pltpu.sync_copy(scale_hbm, wtab)
    wrows = [[jnp.exp(wtab[r, pl.ds(j * L, L)]) for j in range(W // L)]
             for r in range(MAXD)]

    def step(it, bb):
        @pl.when((it >= 2) & valid(it - 2))
        def _():
            out_descr(it - 2, (bb + 2) % NBUF).wait()
        start_in(it + 2, (bb + 2) % NBUF)

        @pl.when(valid(it))
        def _():
            in_descr(it, bb).wait()
            xb = xbufs[bb]

            def group_body(g, _):
                dvec = dball[pl.ds(it * CHUNK + g * L, L)]
                for k in range(L):
                    dr = dvec[k]
                    row = g * L + k
                    for j in range(W // L):
                        sl = pl.ds(j * L, L)
                        w = jnp.where(
                            dr == 0, wrows[0][j],
                            jnp.where(dr == 1, wrows[1][j],
                                      jnp.where(dr == 2, wrows[2][j], wrows[3][j])))
                        xb[row, sl] = xb[row, sl] * w
                return 0

            lax.fori_loop(0, CHUNK // L, group_body, 0)
            out_descr(it, bb).start()

    def ring_body(i, _):
        for bb in range(NBUF):
            step(NBUF * i + bb, bb)
        return 0

    lax.fori_loop(0, ITERS // NBUF, ring_body, 0)

    for it in (ITERS - 2, ITERS - 1):
        @pl.when(valid(it))
        def _(it=it, b=it % NBUF):
            out_descr(it, b).wait()


def kernel(x, d, scale):
    return _scale_degree(x, d.astype(jnp.int32), scale)


# speedup vs baseline: 1.0056x; 1.0056x over previous
import functools

import jax
import jax.numpy as jnp
from jax import lax
from jax.experimental import pallas as pl
from jax.experimental.pallas import tpu as pltpu
from jax.experimental.pallas import tpu_sc as plsc

N = 100000
W = 128
MAXD = 4
L = 16
NC = 2
NS = 16
NW = NC * NS
CHUNK = 160
NBUF = 4
NCHUNKS = N // CHUNK
ITERS = -(-NCHUNKS // NW)
assert ITERS % NBUF == 0

_mesh = plsc.VectorSubcoreMesh(core_axis_name="c", subcore_axis_name="s")


@functools.partial(
    pl.kernel,
    out_type=jax.ShapeDtypeStruct((N, W), jnp.float32),
    mesh=_mesh,
    scratch_types=(
        [pltpu.VMEM((MAXD, W), jnp.float32)]
        + [pltpu.VMEM((CHUNK, W), jnp.float32)] * NBUF
        + [pltpu.VMEM((CHUNK,), jnp.int32)] * NBUF
        + [pltpu.SemaphoreType.DMA] * NBUF
        + [pltpu.SemaphoreType.DMA] * NBUF
    ),
)
def _scale_degree(x_hbm, d_hbm, scale_hbm, out_hbm, wtab, *bufs):
    xbufs = bufs[0:NBUF]
    dbufs = bufs[NBUF:2 * NBUF]
    sin = bufs[2 * NBUF:3 * NBUF]
    sout = bufs[3 * NBUF:4 * NBUF]
    wid = lax.axis_index("s") * NC + lax.axis_index("c")


    def valid(it):
        return (it * NW + wid) < NCHUNKS

    def in_descrs(it, b):
        base = (it * NW + wid) * CHUNK
        return (
            pltpu.make_async_copy(x_hbm.at[pl.ds(base, CHUNK)], xbufs[b], sin[b]),
            pltpu.make_async_copy(d_hbm.at[pl.ds(base, CHUNK)], dbufs[b], sin[b]),
        )

    def out_descr(it, b):
        base = (it * NW + wid) * CHUNK
        return pltpu.make_async_copy(xbufs[b], out_hbm.at[pl.ds(base, CHUNK)], sout[b])

    def start_in(it, b):
        @pl.when(valid(it))
        def _():
            cx, cd = in_descrs(it, b)
            cx.start()
            cd.start()

    start_in(0, 0)
    start_in(1, 1)

    pltpu.sync_copy(scale_hbm, wtab)
    wrows = [[jnp.exp(wtab[r, pl.ds(j * L, L)]) for j in range(W // L)]
             for r in range(MAXD)]

    def step(it, bb):
        @pl.when((it >= 2) & valid(it - 2))
        def _():
            out_descr(it - 2, (bb + 2) % NBUF).wait()
        start_in(it + 2, (bb + 2) % NBUF)

        @pl.when(valid(it))
        def _():
            cx, cd = in_descrs(it, bb)
            cx.wait()
            cd.wait()
            xb, db = xbufs[bb], dbufs[bb]

            def group_body(g, _):
                dvec = db[pl.ds(g * L, L)]
                for k in range(L):
                    dr = dvec[k]
                    row = g * L + k
                    for j in range(W // L):
                        sl = pl.ds(j * L, L)
                        w = jnp.where(
                            dr == 0, wrows[0][j],
                            jnp.where(dr == 1, wrows[1][j],
                                      jnp.where(dr == 2, wrows[2][j], wrows[3][j])))
                        xb[row, sl] = xb[row, sl] * w
                return 0

            lax.fori_loop(0, CHUNK // L, group_body, 0)
            out_descr(it, bb).start()

    def ring_body(i, _):
        for bb in range(NBUF):
            step(NBUF * i + bb, bb)
        return 0

    lax.fori_loop(0, ITERS // NBUF, ring_body, 0)

    for it in (ITERS - 2, ITERS - 1):
        @pl.when(valid(it))
        def _(it=it, b=it % NBUF):
            out_descr(it, b).wait()


def kernel(x, d, scale):
    return _scale_degree(x, d.astype(jnp.int32), scale)


# speedup vs baseline: 1.0334x; 1.0276x over previous
import functools

import jax
import jax.numpy as jnp
from jax import lax
from jax.experimental import pallas as pl
from jax.experimental.pallas import tpu as pltpu
from jax.experimental.pallas import tpu_sc as plsc

N = 100000
W = 128
MAXD = 4
L = 16
NC = 2
NS = 16
NW = NC * NS
CHUNK = 160
NBUF = 4
NCHUNKS = N // CHUNK
ITERS = -(-NCHUNKS // NW)

_mesh = plsc.VectorSubcoreMesh(core_axis_name="c", subcore_axis_name="s")


@functools.partial(
    pl.kernel,
    out_type=jax.ShapeDtypeStruct((N, W), jnp.float32),
    mesh=_mesh,
    scratch_types=[
        pltpu.VMEM((MAXD, W), jnp.float32),
        pltpu.VMEM((NBUF * CHUNK, W), jnp.float32),
        pltpu.VMEM((NBUF * CHUNK,), jnp.int32),
        pltpu.SemaphoreType.DMA((NBUF,)),
        pltpu.SemaphoreType.DMA((NBUF,)),
    ],
)
def _scale_degree(x_hbm, d_hbm, scale_hbm, out_hbm, wtab, xball, dball, sin, sout):
    wid = lax.axis_index("s") * NC + lax.axis_index("c")

    def valid(it):
        return (it * NW + wid) < NCHUNKS

    def in_descrs(it, bb):
        base = (it * NW + wid) * CHUNK
        voff = bb * CHUNK
        return (
            pltpu.make_async_copy(x_hbm.at[pl.ds(base, CHUNK)],
                                  xball.at[pl.ds(voff, CHUNK)], sin.at[bb]),
            pltpu.make_async_copy(d_hbm.at[pl.ds(base, CHUNK)],
                                  dball.at[pl.ds(voff, CHUNK)], sin.at[bb]),
        )

    def out_descr(it, bb):
        base = (it * NW + wid) * CHUNK
        voff = bb * CHUNK
        return pltpu.make_async_copy(xball.at[pl.ds(voff, CHUNK)],
                                     out_hbm.at[pl.ds(base, CHUNK)], sout.at[bb])

    def start_in(it, bb):
        @pl.when(valid(it))
        def _():
            cx, cd = in_descrs(it, bb)
            cx.start()
            cd.start()

    start_in(0, 0)
    start_in(1, 1)

    pltpu.sync_copy(scale_hbm, wtab)
    wrows = [[jnp.exp(wtab[r, pl.ds(j * L, L)]) for j in range(W // L)]
             for r in range(MAXD)]

    def step(it, _):
        bb = it & (NBUF - 1)
        bb2 = (it + 2) & (NBUF - 1)

        @pl.when((it >= 2) & valid(it - 2))
        def _():
            out_descr(it - 2, bb2).wait()
        start_in(it + 2, bb2)

        @pl.when(valid(it))
        def _():
            cx, cd = in_descrs(it, bb)
            cx.wait()
            cd.wait()
            voff = bb * CHUNK

            def group_body(g, _):
                dvec = dball[pl.ds(voff + g * L, L)]
                for k in range(L):
                    dr = dvec[k]
                    row = voff + g * L + k
                    for j in range(W // L):
                        sl = pl.ds(j * L, L)
                        w = jnp.where(
                            dr == 0, wrows[0][j],
                            jnp.where(dr == 1, wrows[1][j],
                                      jnp.where(dr == 2, wrows[2][j], wrows[3][j])))
                        xball[row, sl] = xball[row, sl] * w
                return 0

            lax.fori_loop(0, CHUNK // L, group_body, 0)
            out_descr(it, bb).start()
        return 0

    lax.fori_loop(0, ITERS, step, 0)

    for it in (ITERS - 2, ITERS - 1):
        @pl.when(valid(it))
        def _(it=it, bb=it % NBUF):
            out_descr(it, bb).wait()


def kernel(x, d, scale):
    return _scale_degree(x, d.astype(jnp.int32), scale)
